# baseline (speedup 1.0000x reference)
import jax
import jax.numpy as jnp
from jax import lax
from jax.experimental import pallas as pl
from jax.experimental.pallas import tpu as pltpu
from jax.experimental.pallas import tpu_sc as plsc

B, C, T, D, V = 16, 8, 4096, 64, 2048
K = 64
NC, NS = 2, 16
NW = NC * NS
TOK_PER_W = (B * T) // NW
CHUNKS = TOK_PER_W // K


def _embed_body(x_hbm, emb_hbm, out_hbm, idx_v, acc0, acc1, outv0, outv1,
                sg0, sg1, so0, so1):
    cid = lax.axis_index("core")
    sid = lax.axis_index("sub")
    wid = sid * NC + cid
    b = wid // 2
    t_half = (wid % 2) * TOK_PER_W
    acc = (acc0, acc1)
    outv = (outv0, outv1)
    sg = (sg0, sg1)
    so = (so0, so1)

    for c in range(C):
        pltpu.async_copy(x_hbm.at[b, c, pl.ds(t_half, TOK_PER_W)],
                         idx_v.at[c], sg0)
    for c in range(C):
        pltpu.make_async_copy(x_hbm.at[b, c, pl.ds(t_half, TOK_PER_W)],
                              idx_v.at[c], sg0).wait()

    def off_body(j, carry):
        sl = pl.ds(j * 16, 16)
        for c in range(1, C):
            idx_v[c, sl] = idx_v[c, sl] + (c * V)
        return carry
    lax.fori_loop(0, TOK_PER_W // 16, off_body, 0, unroll=2)

    def zero_acc(p):
        def z_body(k, carry):
            for dd in range(D // 16):
                acc[p][k, pl.ds(dd * 16, 16)] = jnp.zeros((16,), jnp.float32)
            return carry
        lax.fori_loop(0, K, z_body, 0, unroll=4)

    def fire_gathers(i, p):
        loc = i * K
        for c in range(C):
            pltpu.async_copy(emb_hbm.at[idx_v.at[c, pl.ds(loc, K)]],
                             acc[p], sg[p], add=True)

    def drain_gathers(i, p):
        loc = i * K
        for c in range(C):
            pltpu.make_async_copy(emb_hbm.at[idx_v.at[c, pl.ds(loc, K)]],
                                  acc[p], sg[p]).wait()

    def scale(p):
        def tok_body(k, carry):
            for dd in range(D // 16):
                sl = pl.ds(dd * 16, 16)
                outv[p][k, sl] = acc[p][k, sl] * (1.0 / C)
            return carry
        lax.fori_loop(0, K, tok_body, 0, unroll=4)

    def fire_out(i, p):
        pltpu.async_copy(outv[p], out_hbm.at[b, pl.ds(t_half + i * K, K)],
                         so[p])

    def wait_out(i, p):
        pltpu.make_async_copy(outv[p], out_hbm.at[b, pl.ds(t_half + i * K, K)],
                              so[p]).wait()

    zero_acc(0)
    zero_acc(1)
    fire_gathers(0, 0)

    def pair_body(i, carry):
        ii = 2 * i
        fire_gathers(ii + 1, 1)
        drain_gathers(ii, 0)
        @pl.when(i > 0)
        def _():
            wait_out(ii - 2, 0)
        scale(0)
        zero_acc(0)
        fire_out(ii, 0)
        @pl.when(ii + 2 < CHUNKS)
        def _():
            fire_gathers(ii + 2, 0)
        drain_gathers(ii + 1, 1)
        @pl.when(i > 0)
        def _():
            wait_out(ii - 1, 1)
        scale(1)
        zero_acc(1)
        fire_out(ii + 1, 1)
        return carry

    lax.fori_loop(0, CHUNKS // 2, pair_body, 0)
    wait_out(CHUNKS - 2, 0)
    wait_out(CHUNKS - 1, 1)


_mesh = plsc.VectorSubcoreMesh(
    core_axis_name="core", subcore_axis_name="sub",
    num_cores=NC, num_subcores=NS)

_embed = pl.kernel(
    _embed_body,
    out_type=jax.ShapeDtypeStruct((B, T, D), jnp.float32),
    mesh=_mesh,
    scratch_types=[
        pltpu.VMEM((C, TOK_PER_W), jnp.int32),
        pltpu.VMEM((K, D), jnp.float32),
        pltpu.VMEM((K, D), jnp.float32),
        pltpu.VMEM((K, D), jnp.float32),
        pltpu.VMEM((K, D), jnp.float32),
        pltpu.SemaphoreType.DMA,
        pltpu.SemaphoreType.DMA,
        pltpu.SemaphoreType.DMA,
        pltpu.SemaphoreType.DMA,
    ],
    compiler_params=pltpu.CompilerParams(use_tc_tiling_on_sc=False),
)


def kernel(x, emb):
    return _embed(x.astype(jnp.int32), emb)


# speedup vs baseline: 1.0697x; 1.0697x over previous
import jax
import jax.numpy as jnp
from jax import lax
from jax.experimental import pallas as pl
from jax.experimental.pallas import tpu as pltpu
from jax.experimental.pallas import tpu_sc as plsc

B, C, T, D, V = 16, 8, 4096, 64, 2048
K = 64
NC, NS = 2, 16
NW = NC * NS
TOK_PER_W = (B * T) // NW
CHUNKS = TOK_PER_W // K


def _embed_body(x_hbm, emb_hbm, out_hbm, idx_v, rows0, rows1, outv0, outv1,
                sg0, sg1, so0, so1):
    cid = lax.axis_index("core")
    sid = lax.axis_index("sub")
    wid = sid * NC + cid
    b = wid // 2
    t_half = (wid % 2) * TOK_PER_W
    rows = (rows0, rows1)
    outv = (outv0, outv1)
    sg = (sg0, sg1)
    so = (so0, so1)

    for c in range(C):
        pltpu.async_copy(x_hbm.at[b, c, pl.ds(t_half, TOK_PER_W)],
                         idx_v.at[c], sg0)
    for c in range(C):
        pltpu.make_async_copy(x_hbm.at[b, c, pl.ds(t_half, TOK_PER_W)],
                              idx_v.at[c], sg0).wait()

    def off_body(j, carry):
        sl = pl.ds(j * 16, 16)
        for c in range(1, C):
            idx_v[c, sl] = idx_v[c, sl] + (c * V)
        return carry
    lax.fori_loop(0, TOK_PER_W // 16, off_body, 0, unroll=2)

    def fire_gathers(i, p):
        loc = i * K
        for c in range(C):
            pltpu.async_copy(
                emb_hbm.at[idx_v.at[c, pl.ds(loc, K)]], rows[p].at[c], sg[p])

    def drain_gathers(i, p):
        loc = i * K
        for c in range(C):
            pltpu.make_async_copy(
                emb_hbm.at[idx_v.at[c, pl.ds(loc, K)]], rows[p].at[c],
                sg[p]).wait()

    def accum(p):
        def tok_body(k, carry):
            acc = [None] * (D // 16)
            for c in range(C):
                for g in range(D // 32):
                    v = rows[p][c, k, pl.ds(g * 32, 32)]
                    a, bb = plsc.unpack(
                        v, format=plsc.PackFormat.INTERLEAVED)
                    if c == 0:
                        acc[2 * g], acc[2 * g + 1] = a, bb
                    else:
                        acc[2 * g] = acc[2 * g] + a
                        acc[2 * g + 1] = acc[2 * g + 1] + bb
            for dd in range(D // 16):
                outv[p][k, pl.ds(dd * 16, 16)] = acc[dd] * (1.0 / C)
            return carry
        lax.fori_loop(0, K, tok_body, 0, unroll=2)

    def fire_out(i, p):
        pltpu.async_copy(outv[p], out_hbm.at[b, pl.ds(t_half + i * K, K)],
                         so[p])

    def wait_out(i, p):
        pltpu.make_async_copy(outv[p], out_hbm.at[b, pl.ds(t_half + i * K, K)],
                              so[p]).wait()

    fire_gathers(0, 0)

    def pair_body(i, carry):
        ii = 2 * i
        fire_gathers(ii + 1, 1)
        drain_gathers(ii, 0)
        @pl.when(i > 0)
        def _():
            wait_out(ii - 2, 0)
        accum(0)
        fire_out(ii, 0)
        @pl.when(ii + 2 < CHUNKS)
        def _():
            fire_gathers(ii + 2, 0)
        drain_gathers(ii + 1, 1)
        @pl.when(i > 0)
        def _():
            wait_out(ii - 1, 1)
        accum(1)
        fire_out(ii + 1, 1)
        return carry

    lax.fori_loop(0, CHUNKS // 2, pair_body, 0)
    wait_out(CHUNKS - 2, 0)
    wait_out(CHUNKS - 1, 1)


_mesh = plsc.VectorSubcoreMesh(
    core_axis_name="core", subcore_axis_name="sub",
    num_cores=NC, num_subcores=NS)

_embed = pl.kernel(
    _embed_body,
    out_type=jax.ShapeDtypeStruct((B, T, D), jnp.float32),
    mesh=_mesh,
    scratch_types=[
        pltpu.VMEM((C, TOK_PER_W), jnp.int32),
        pltpu.VMEM((C, K, D), jnp.bfloat16),
        pltpu.VMEM((C, K, D), jnp.bfloat16),
        pltpu.VMEM((K, D), jnp.float32),
        pltpu.VMEM((K, D), jnp.float32),
        pltpu.SemaphoreType.DMA,
        pltpu.SemaphoreType.DMA,
        pltpu.SemaphoreType.DMA,
        pltpu.SemaphoreType.DMA,
    ],
    compiler_params=pltpu.CompilerParams(
        use_tc_tiling_on_sc=False, needs_layout_passes=False),
)


def kernel(x, emb):
    emb_bf = (emb.astype(jnp.bfloat16)
              .reshape(C * V, D // 32, 2, 16)
              .transpose(0, 1, 3, 2)
              .reshape(C * V, D))
    return _embed(x.astype(jnp.int32), emb_bf)


# speedup vs baseline: 1.0711x; 1.0013x over previous
import jax
import jax.numpy as jnp
from jax import lax
from jax.experimental import pallas as pl
from jax.experimental.pallas import tpu as pltpu
from jax.experimental.pallas import tpu_sc as plsc

B, C, T, D, V = 16, 8, 4096, 64, 2048
K = 64
NC, NS = 2, 16
NW = NC * NS
TOK_PER_W = (B * T) // NW
CHUNKS = TOK_PER_W // K


def _embed_body(x_hbm, emb_hbm, out_hbm, idx_v, rows0, rows1, outv0, outv1,
                sg0, sg1, so0, so1):
    cid = lax.axis_index("core")
    sid = lax.axis_index("sub")
    wid = sid * NC + cid
    b = wid // 2
    t_half = (wid % 2) * TOK_PER_W
    rows = (rows0, rows1)
    outv = (outv0, outv1)
    sg = (sg0, sg1)
    so = (so0, so1)

    for c in range(C):
        pltpu.async_copy(x_hbm.at[b, c, pl.ds(t_half, TOK_PER_W)],
                         idx_v.at[c], sg0)
    for c in range(C):
        pltpu.make_async_copy(x_hbm.at[b, c, pl.ds(t_half, TOK_PER_W)],
                              idx_v.at[c], sg0).wait()

    def off_body(j, carry):
        sl = pl.ds(j * 16, 16)
        for c in range(1, C):
            idx_v[c, sl] = idx_v[c, sl] + (c * V)
        return carry
    lax.fori_loop(0, TOK_PER_W // 16, off_body, 0, unroll=2)

    def fire_gathers(i, p):
        loc = i * K
        for c in range(C):
            pltpu.async_copy(
                emb_hbm.at[idx_v.at[c, pl.ds(loc, K)]], rows[p].at[c], sg[p])

    def drain_gathers(i, p):
        loc = i * K
        for c in range(C):
            pltpu.make_async_copy(
                emb_hbm.at[idx_v.at[c, pl.ds(loc, K)]], rows[p].at[c],
                sg[p]).wait()

    def accum(p):
        def tok_body(k, carry):
            acc = [None] * (D // 16)
            for c in range(C):
                for g in range(D // 32):
                    v = rows[p][c, k, pl.ds(g * 32, 32)]
                    a, bb = plsc.unpack(
                        v, format=plsc.PackFormat.INTERLEAVED)
                    if c == 0:
                        acc[2 * g], acc[2 * g + 1] = a, bb
                    else:
                        acc[2 * g] = acc[2 * g] + a
                        acc[2 * g + 1] = acc[2 * g + 1] + bb
            for dd in range(D // 16):
                outv[p][pl.ds(k * D + dd * 16, 16)] = acc[dd] * (1.0 / C)
            return carry
        lax.fori_loop(0, K, tok_body, 0, unroll=2)

    def fire_out(i, p):
        base = (b * T + t_half + i * K) * D
        pltpu.async_copy(outv[p], out_hbm.at[pl.ds(base, K * D)], so[p])

    def wait_out(i, p):
        base = (b * T + t_half + i * K) * D
        pltpu.make_async_copy(outv[p], out_hbm.at[pl.ds(base, K * D)],
                              so[p]).wait()

    fire_gathers(0, 0)

    def pair_body(i, carry):
        ii = 2 * i
        fire_gathers(ii + 1, 1)
        drain_gathers(ii, 0)
        @pl.when(i > 0)
        def _():
            wait_out(ii - 2, 0)
        accum(0)
        fire_out(ii, 0)
        @pl.when(ii + 2 < CHUNKS)
        def _():
            fire_gathers(ii + 2, 0)
        drain_gathers(ii + 1, 1)
        @pl.when(i > 0)
        def _():
            wait_out(ii - 1, 1)
        accum(1)
        fire_out(ii + 1, 1)
        return carry

    lax.fori_loop(0, CHUNKS // 2, pair_body, 0)
    wait_out(CHUNKS - 2, 0)
    wait_out(CHUNKS - 1, 1)


_mesh = plsc.VectorSubcoreMesh(
    core_axis_name="core", subcore_axis_name="sub",
    num_cores=NC, num_subcores=NS)

_embed = pl.kernel(
    _embed_body,
    out_type=jax.ShapeDtypeStruct((B * T * D,), jnp.float32),
    mesh=_mesh,
    scratch_types=[
        pltpu.VMEM((C, TOK_PER_W), jnp.int32),
        pltpu.VMEM((C, K, D), jnp.bfloat16),
        pltpu.VMEM((C, K, D), jnp.bfloat16),
        pltpu.VMEM((K * D,), jnp.float32),
        pltpu.VMEM((K * D,), jnp.float32),
        pltpu.SemaphoreType.DMA,
        pltpu.SemaphoreType.DMA,
        pltpu.SemaphoreType.DMA,
        pltpu.SemaphoreType.DMA,
    ],
    compiler_params=pltpu.CompilerParams(
        use_tc_tiling_on_sc=False, needs_layout_passes=False),
)


def kernel(x, emb):
    emb_bf = (emb.astype(jnp.bfloat16)
              .reshape(C * V, D // 32, 2, 16)
              .transpose(0, 1, 3, 2)
              .reshape(C * V, D))
    return _embed(x.astype(jnp.int32), emb_bf).reshape(B, T, D)
